# Initial kernel scaffold; baseline (speedup 1.0000x reference)
import jax
import jax.numpy as jnp
from jax.experimental import pallas as pl


def kernel(xyz, params):
    raise NotImplementedError("write your pallas kernel here")


# speedup vs baseline: 20.7732x; 20.7732x over previous
import functools

import jax
import jax.numpy as jnp
from jax.experimental import pallas as pl

_CFGS = [
    (1024, 32, 64, 2), (1024, 64, 64, 4), (1024, 64, 128, 2),
    (1024, 128, 128, 4), (256, 128, 256, 2), (256, 256, 256, 4),
    (64, 256, 512, 2), (64, 512, 512, 4),
]
_K = 20


def _knn_body(pts_ref, idx_ref, *, n, k):
    x = pts_ref[0]
    sq = jnp.sum(x * x, axis=1)
    inner = jax.lax.dot_general(
        x, x, (((1,), (1,)), ((), ())),
        preferred_element_type=jnp.float32)
    dist = (sq[:, None] - 2.0 * inner) + sq[None, :]
    iota = jax.lax.broadcasted_iota(jnp.int32, (n, n), 1)
    for kk in range(k):
        m = jnp.min(dist, axis=1, keepdims=True)
        eq = dist == m
        cand = jnp.where(eq, iota, n)
        sel = jnp.min(cand, axis=1)
        idx_ref[0, kk, :] = sel
        dist = jnp.where(iota == sel[:, None], 1e30, dist)


def _knn(pts, k):
    b, n, _ = pts.shape
    return pl.pallas_call(
        functools.partial(_knn_body, n=n, k=k),
        grid=(b,),
        in_specs=[pl.BlockSpec((1, n, 3), lambda i: (i, 0, 0))],
        out_specs=pl.BlockSpec((1, k, n), lambda i: (i, 0, 0)),
        out_shape=jax.ShapeDtypeStruct((b, k, n), jnp.int32),
    )(pts)


def _gather_max(idx_ref, table, n, k):
    iota = jax.lax.broadcasted_iota(jnp.int32, (n, n), 1)
    m = None
    for kk in range(k):
        sel = idx_ref[0, kk, :]
        oh = (sel[:, None] == iota).astype(jnp.float32)
        g = jax.lax.dot_general(
            oh, table, (((1,), (0,)), ((), ())),
            preferred_element_type=jnp.float32)
        m = g if m is None else jnp.maximum(m, g)
    return m


def _lpfa_body(pts_ref, idx_ref, wsum_ref, wdiff_ref, out_ref, *, n, k):
    x = pts_ref[0]
    p = x @ wsum_ref[...]
    v = x @ wdiff_ref[...]
    m = _gather_max(idx_ref, p, n, k)
    out_ref[0] = jax.nn.relu(m + v)


def _lpfa(pts, idx, wsum, wdiff):
    b, n, _ = pts.shape
    c = wsum.shape[1]
    return pl.pallas_call(
        functools.partial(_lpfa_body, n=n, k=_K),
        grid=(b,),
        in_specs=[
            pl.BlockSpec((1, n, 3), lambda i: (i, 0, 0)),
            pl.BlockSpec((1, _K, n), lambda i: (i, 0, 0)),
            pl.BlockSpec(wsum.shape, lambda i: (0, 0)),
            pl.BlockSpec(wdiff.shape, lambda i: (0, 0)),
        ],
        out_specs=pl.BlockSpec((1, n, c), lambda i: (i, 0, 0)),
        out_shape=jax.ShapeDtypeStruct((b, n, c), jnp.float32),
    )(pts, idx, wsum, wdiff)


def _cic_body(feat_ref, idx_ref, w1_ref, w2a_ref, w2d_ref, wsc_ref, out_ref,
              *, n, k):
    f = feat_ref[0]
    h = jax.nn.relu(f @ w1_ref[...])
    u = h @ w2a_ref[...]
    v = h @ w2d_ref[...]
    s = f @ wsc_ref[...]
    m = _gather_max(idx_ref, u, n, k)
    out_ref[0] = jax.nn.relu(jax.nn.relu(m + v) + s)


def _cic(feat, idx, w1, w2a, w2d, wsc):
    b, n, _ = feat.shape
    cout = wsc.shape[1]
    return pl.pallas_call(
        functools.partial(_cic_body, n=n, k=_K),
        grid=(b,),
        in_specs=[
            pl.BlockSpec((1, n, feat.shape[2]), lambda i: (i, 0, 0)),
            pl.BlockSpec((1, _K, n), lambda i: (i, 0, 0)),
            pl.BlockSpec(w1.shape, lambda i: (0, 0)),
            pl.BlockSpec(w2a.shape, lambda i: (0, 0)),
            pl.BlockSpec(w2d.shape, lambda i: (0, 0)),
            pl.BlockSpec(wsc.shape, lambda i: (0, 0)),
        ],
        out_specs=pl.BlockSpec((1, n, cout), lambda i: (i, 0, 0)),
        out_shape=jax.ShapeDtypeStruct((b, n, cout), jnp.float32),
    )(feat, idx, w1, w2a, w2d, wsc)


def _head_body(feat_ref, c0_ref, c1_ref, c2_ref, b2_ref,
               logits_ref, latent_ref):
    f = feat_ref[...]
    b, n, c = f.shape
    h = jax.nn.relu(
        jnp.reshape(f, (b * n, c)) @ c0_ref[...])
    h = jnp.reshape(h, (b, n, h.shape[1]))
    mx = jnp.max(h, axis=1)
    av = jnp.sum(h, axis=1) * (1.0 / n)
    latent = jnp.concatenate([mx, av], axis=1)
    x1 = jax.nn.relu(latent @ c1_ref[...])
    logits_ref[...] = x1 @ c2_ref[...] + b2_ref[...]
    latent_ref[...] = latent


def _head(feat, c0, c1, c2, b2):
    b = feat.shape[0]
    return pl.pallas_call(
        _head_body,
        out_shape=(
            jax.ShapeDtypeStruct((b, c2.shape[1]), jnp.float32),
            jax.ShapeDtypeStruct((b, 2 * c0.shape[1]), jnp.float32),
        ),
    )(feat, c0, c1, c2, b2.reshape(1, -1))


def kernel(xyz, params):
    pts = jnp.swapaxes(xyz, 1, 2)
    idx_by_n = {1024: _knn(pts, _K),
                256: _knn(pts[:, ::4], _K),
                64: _knn(pts[:, ::16], _K)}

    lw = params['lpfa_W']
    wsum = lw[0:3] + lw[3:6]
    wdiff = lw[6:9] - lw[0:3]
    feat = _lpfa(pts, idx_by_n[1024], wsum, wdiff)

    cur_n = 1024
    for (npoint, cin, cout, ratio), p in zip(_CFGS, params['cic']):
        mid = cout // ratio
        if npoint < cur_n:
            feat = feat[:, ::cur_n // npoint]
            cur_n = npoint
        w2a = p['W2'][:mid]
        w2d = p['W2'][mid:] - w2a
        feat = _cic(feat, idx_by_n[cur_n], p['W1'], w2a, w2d, p['Wsc'])

    return _head(feat, params['conv0_W'], params['conv1_W'],
                 params['conv2_W'], params['conv2_b'])
